# baseline (speedup 1.0000x reference)
import functools
import math

import ml_dtypes
import numpy as np
import jax
import jax.numpy as jnp
from jax import lax
from jax.experimental import pallas as pl
from jax.experimental.pallas import tpu as pltpu
from jax.experimental.pallas import tpu_sc as plsc

D_MODEL = 1024
MAX_LEN = 8192
SCALE = math.sqrt(D_MODEL)

NUM_CORES = 2
NUM_SUBCORES = 16
NW = NUM_CORES * NUM_SUBCORES
LANES = 16

NSLOT = 4
AHEAD = 2
CP = 8
CPI = 16
UNROLL = 4


def _sinusoidal_pos_encoding_np(max_len, d_model):
    pos = np.arange(max_len, dtype=np.float32)[:, None]
    i = np.arange(0, d_model, 2, dtype=np.float32)[None, :]
    angle = pos / np.power(10000.0, i / d_model)
    enc = np.zeros((max_len, d_model), dtype=np.float32)
    enc[:, 0::2] = np.sin(angle)
    enc[:, 1::2] = np.cos(angle)
    return enc


def _pos_bf16_packed(max_len, d_model):
    p = _sinusoidal_pos_encoding_np(max_len, d_model)
    b = p.astype(ml_dtypes.bfloat16).view(np.uint16)
    b = b.reshape(max_len, d_model // 32, 32)
    lo = b[:, :, :16].astype(np.uint32)
    hi = b[:, :, 16:].astype(np.uint32)
    return (lo | (hi << 16)).reshape(max_len, d_model // 2).view(np.int32)


_POS_PACKED_NP = _pos_bf16_packed(MAX_LEN, D_MODEL)


def _make_kernel(BATCH, S):
    PPW = S // NW
    NPAIR = PPW // CPI
    NITEM = NPAIR * BATCH
    assert BATCH == 4 and NSLOT == BATCH and NITEM % (2 * BATCH) == 0

    mesh = plsc.VectorSubcoreMesh(core_axis_name="c", subcore_axis_name="s")

    @functools.partial(
        pl.kernel,
        mesh=mesh,
        out_type=jax.ShapeDtypeStruct((BATCH * S, D_MODEL), jnp.float32),
        scratch_types=[
            pltpu.VMEM((BATCH * PPW,), jnp.int32),
            pltpu.VMEM((NSLOT, CPI, D_MODEL), jnp.float32),
            pltpu.VMEM((2, CPI, D_MODEL // 2), jnp.int32),
            pltpu.SemaphoreType.DMA((NSLOT,)),
            pltpu.SemaphoreType.DMA((2,)),
            pltpu.SemaphoreType.DMA((NSLOT,)),
        ],
    )
    def emb(x_hbm, tab_hbm, pos_hbm, out_hbm, idx_v, row_v, pos_v,
            gsem, psem, osem):
        wid = lax.axis_index("s") * NUM_CORES + lax.axis_index("c")
        p_base = wid * PPW

        for b in range(BATCH):
            pltpu.sync_copy(
                x_hbm.at[pl.ds(b * S + p_base, PPW)],
                idx_v.at[pl.ds(b * PPW, PPW)])

        def issue_gather(q, b, slot):
            for h in range(2):
                pltpu.async_copy(
                    tab_hbm.at[idx_v.at[
                        pl.ds(b * PPW + q * CPI + h * CP, CP)]],
                    row_v.at[slot].at[pl.ds(h * CP, CP)], gsem.at[slot])

        def wait_gather(slot):
            for h in range(2):
                pltpu.make_async_copy(
                    tab_hbm.at[idx_v.at[pl.ds(0, CP)]],
                    row_v.at[slot].at[pl.ds(h * CP, CP)],
                    gsem.at[slot]).wait()

        def issue_pos(q, pslot):
            pltpu.async_copy(
                pos_hbm.at[pl.ds(p_base + q * CPI, CPI)], pos_v.at[pslot],
                psem.at[pslot])

        def wait_pos(pslot):
            pltpu.make_async_copy(
                pos_hbm.at[pl.ds(0, CPI)], pos_v.at[pslot],
                psem.at[pslot]).wait()

        def issue_out(q, b, slot):
            pltpu.async_copy(
                row_v.at[slot],
                out_hbm.at[pl.ds(b * S + p_base + q * CPI, CPI)],
                osem.at[slot])

        def wait_out(slot):
            pltpu.make_async_copy(
                row_v.at[slot], out_hbm.at[pl.ds(0, CPI)],
                osem.at[slot]).wait()

        def compute(slot, pslot):
            @pl.loop(0, CPI)
            def _row(r):
                rr = row_v.at[slot].at[r]
                pp = pos_v.at[pslot].at[r]

                @plsc.parallel_loop(0, D_MODEL // (2 * LANES),
                                    unroll=UNROLL)
                def _vec(k):
                    k0 = k * 2 * LANES
                    pw = pp[pl.ds(k * LANES, LANES)]
                    pa = lax.bitcast_convert_type(
                        pw << 16, jnp.float32)
                    pb = lax.bitcast_convert_type(
                        pw & jnp.int32(-65536), jnp.float32)
                    sl0 = pl.ds(k0, LANES)
                    sl1 = pl.ds(k0 + LANES, LANES)
                    rr[sl0] = rr[sl0] * SCALE + pa
                    rr[sl1] = rr[sl1] * SCALE + pb

        issue_pos(0, 0)
        for j in range(AHEAD):
            issue_gather(j // BATCH, j % BATCH, j)

        @pl.loop(0, NITEM, step=2 * BATCH)
        def _grp(ii):
            q0 = ii >> 2
            for u in range(2 * BATCH):
                i = ii + u
                slot = u % NSLOT
                b = u % BATCH
                pslot = u // BATCH
                q = q0 + pslot

                if u == 0:
                    wait_pos(0)
                    issue_pos(q0 + 1, 1)
                if u == BATCH:
                    wait_pos(1)

                    @pl.when(q0 + 2 < NPAIR)
                    def _():
                        issue_pos(q0 + 2, 0)

                s2 = (u + AHEAD) % NSLOT
                b_a = (u + AHEAD) % BATCH
                dq_a = (b + AHEAD) // BATCH

                @pl.when(i >= NSLOT - AHEAD)
                def _():
                    wait_out(s2)

                @pl.when(i + AHEAD < NITEM)
                def _():
                    issue_gather(q + dq_a, b_a, s2)

                wait_gather(slot)
                compute(slot, pslot)
                issue_out(q, b, slot)

        for j in range(NSLOT - AHEAD):
            wait_out((NITEM - (NSLOT - AHEAD) + j) % NSLOT)

    return emb


def kernel(x, table):
    B_, S_ = x.shape
    flat_x = x.reshape(-1).astype(jnp.int32)
    emb = _make_kernel(B_, S_)
    out = emb(flat_x, table, jnp.asarray(_POS_PACKED_NP))
    return out.reshape(B_, S_, D_MODEL)


# speedup vs baseline: 1.0105x; 1.0105x over previous
import functools
import math

import ml_dtypes
import numpy as np
import jax
import jax.numpy as jnp
from jax import lax
from jax.experimental import pallas as pl
from jax.experimental.pallas import tpu as pltpu
from jax.experimental.pallas import tpu_sc as plsc

D_MODEL = 1024
MAX_LEN = 8192
SCALE = math.sqrt(D_MODEL)

NUM_CORES = 2
NUM_SUBCORES = 16
NW = NUM_CORES * NUM_SUBCORES
LANES = 16

NSLOT = 8
AHEAD = 4
CP = 8
UNROLL = 4


def _sinusoidal_pos_encoding_np(max_len, d_model):
    pos = np.arange(max_len, dtype=np.float32)[:, None]
    i = np.arange(0, d_model, 2, dtype=np.float32)[None, :]
    angle = pos / np.power(10000.0, i / d_model)
    enc = np.zeros((max_len, d_model), dtype=np.float32)
    enc[:, 0::2] = np.sin(angle)
    enc[:, 1::2] = np.cos(angle)
    return enc


def _pos_bf16_packed(max_len, d_model):
    p = _sinusoidal_pos_encoding_np(max_len, d_model)
    b = p.astype(ml_dtypes.bfloat16).view(np.uint16)
    b = b.reshape(max_len, d_model // 32, 32)
    lo = b[:, :, :16].astype(np.uint32)
    hi = b[:, :, 16:].astype(np.uint32)
    return (lo | (hi << 16)).reshape(max_len, d_model // 2).view(np.int32)


_POS_PACKED_NP = _pos_bf16_packed(MAX_LEN, D_MODEL)


def _make_kernel(BATCH, S):
    PPW = S // NW
    NCH = PPW // CP
    NITEM = NCH * BATCH
    assert BATCH == 4 and NSLOT == 2 * BATCH and NITEM % NSLOT == 0

    mesh = plsc.VectorSubcoreMesh(core_axis_name="c", subcore_axis_name="s")

    @functools.partial(
        pl.kernel,
        mesh=mesh,
        out_type=jax.ShapeDtypeStruct((BATCH * S, D_MODEL), jnp.float32),
        scratch_types=[
            pltpu.VMEM((BATCH * PPW,), jnp.int32),
            pltpu.VMEM((NSLOT, CP, D_MODEL), jnp.float32),
            pltpu.VMEM((2, CP, D_MODEL // 2), jnp.int32),
            pltpu.SemaphoreType.DMA((NSLOT,)),
            pltpu.SemaphoreType.DMA((2,)),
            pltpu.SemaphoreType.DMA((NSLOT,)),
        ],
    )
    def emb(x_hbm, tab_hbm, pos_hbm, out_hbm, idx_v, row_v, pos_v,
            gsem, psem, osem):
        wid = lax.axis_index("s") * NUM_CORES + lax.axis_index("c")
        p_base = wid * PPW

        for b in range(BATCH):
            pltpu.sync_copy(
                x_hbm.at[pl.ds(b * S + p_base, PPW)],
                idx_v.at[pl.ds(b * PPW, PPW)])

        def issue_gather(c, b, slot):
            pltpu.async_copy(
                tab_hbm.at[idx_v.at[pl.ds(b * PPW + c * CP, CP)]],
                row_v.at[slot], gsem.at[slot])

        def wait_gather(slot):
            pltpu.make_async_copy(
                tab_hbm.at[idx_v.at[pl.ds(0, CP)]], row_v.at[slot],
                gsem.at[slot]).wait()

        def issue_pos(c, pslot):
            pltpu.async_copy(
                pos_hbm.at[pl.ds(p_base + c * CP, CP)], pos_v.at[pslot],
                psem.at[pslot])

        def wait_pos(pslot):
            pltpu.make_async_copy(
                pos_hbm.at[pl.ds(0, CP)], pos_v.at[pslot],
                psem.at[pslot]).wait()

        def issue_out(c, b, slot):
            pltpu.async_copy(
                row_v.at[slot],
                out_hbm.at[pl.ds(b * S + p_base + c * CP, CP)],
                osem.at[slot])

        def wait_out(slot):
            pltpu.make_async_copy(
                row_v.at[slot], out_hbm.at[pl.ds(0, CP)],
                osem.at[slot]).wait()

        def compute(slot, pslot):
            @pl.loop(0, CP)
            def _row(r):
                rr = row_v.at[slot].at[r]
                pp = pos_v.at[pslot].at[r]

                @plsc.parallel_loop(0, D_MODEL // (2 * LANES),
                                    unroll=UNROLL)
                def _vec(k):
                    k0 = k * 2 * LANES
                    pw = pp[pl.ds(k * LANES, LANES)]
                    pa = lax.bitcast_convert_type(
                        pw << 16, jnp.float32)
                    pb = lax.bitcast_convert_type(
                        pw & jnp.int32(-65536), jnp.float32)
                    sl0 = pl.ds(k0, LANES)
                    sl1 = pl.ds(k0 + LANES, LANES)
                    rr[sl0] = rr[sl0] * SCALE + pa
                    rr[sl1] = rr[sl1] * SCALE + pb

        issue_pos(0, 0)
        for b in range(BATCH):
            issue_gather(0, b, b)

        @pl.loop(0, NITEM, step=NSLOT)
        def _grp(ii):
            c0 = ii >> 2
            for u in range(NSLOT):
                i = ii + u
                slot = u
                b = u % BATCH
                pslot = u // BATCH
                c = c0 + pslot

                if u == 0:
                    wait_pos(0)
                    issue_pos(c0 + 1, 1)
                if u == BATCH:
                    wait_pos(1)

                    @pl.when(c0 + 2 < NCH)
                    def _():
                        issue_pos(c0 + 2, 0)

                s4 = (u + AHEAD) % NSLOT

                @pl.when(i >= AHEAD)
                def _():
                    wait_out(s4)

                @pl.when(i + AHEAD < NITEM)
                def _():
                    issue_gather(c + 1, b, s4)

                wait_gather(slot)
                compute(slot, pslot)
                issue_out(c, b, slot)

        for u in range(AHEAD):
            wait_out((NITEM - AHEAD + u) % NSLOT)

    return emb


def kernel(x, table):
    B_, S_ = x.shape
    flat_x = x.reshape(-1).astype(jnp.int32)
    emb = _make_kernel(B_, S_)
    out = emb(flat_x, table, jnp.asarray(_POS_PACKED_NP))
    return out.reshape(B_, S_, D_MODEL)
